# Initial kernel scaffold; baseline (speedup 1.0000x reference)
import jax
import jax.numpy as jnp
from jax.experimental import pallas as pl


def kernel(feat, edge_index, W1, b1, W2, b2):
    raise NotImplementedError("write your pallas kernel here")


# speedup vs baseline: 6.7052x; 6.7052x over previous
import functools

import jax
import jax.numpy as jnp
from jax import lax
from jax.experimental import pallas as pl
from jax.experimental.pallas import tpu as pltpu
from jax.experimental.pallas import tpu_sc as plsc

N = 10000
E = 320000
D = 128

NC = 2
NS = 16
NW = NC * NS
CHUNK = 80
EPT = E // NW
ITERS = EPT // CHUNK
RA = 632
RB = N - (NS - 1) * RA


def _mlp_body(x_ref, w1_ref, w2_ref, o_ref):
    h = lax.dot_general(x_ref[...], w1_ref[...], (((1,), (1,)), ((), ())),
                        preferred_element_type=jnp.float32)
    o_ref[...] = lax.dot_general(h, w2_ref[...], (((1,), (1,)), ((), ())),
                                 preferred_element_type=jnp.float32)


def _mlp(x, W1, W2):
    return pl.pallas_call(
        _mlp_body,
        out_shape=jax.ShapeDtypeStruct((N, D), jnp.float32),
    )(x, W1, W2)


def _combine_body(p_ref, b1_ref, w2_ref, b2_ref, o_ref):
    c = lax.dot_general(b1_ref[...], w2_ref[...], (((1,), (1,)), ((), ())),
                        preferred_element_type=jnp.float32) + b2_ref[...]
    o_ref[...] = p_ref[:N] + p_ref[N:] + c


def _combine(p, b1row, W2, b2row):
    return pl.pallas_call(
        _combine_body,
        out_shape=jax.ShapeDtypeStruct((N, D), jnp.float32),
    )(p, b1row, W2, b2row)


_MESH = plsc.VectorSubcoreMesh(core_axis_name="c", subcore_axis_name="s")


@functools.partial(
    pl.kernel,
    mesh=_MESH,
    out_type=jax.ShapeDtypeStruct((NC * N, D), jnp.float32),
    scratch_types=[
        pltpu.VMEM((ITERS, CHUNK), jnp.int32),
        pltpu.VMEM((ITERS, CHUNK), jnp.int32),
        pltpu.VMEM((CHUNK, D), jnp.float32),
        pltpu.VMEM_SHARED((N, D), jnp.float32),
        pltpu.SemaphoreType.DMA,
    ],
)
def _spmm(src_hbm, dst_hbm, g_hbm, z_hbm, out_hbm, sidx, didx, rows, acc, sem):
    cid = lax.axis_index("c")
    sid = lax.axis_index("s")
    wid = cid * NS + sid

    pltpu.sync_copy(src_hbm.at[wid], sidx)
    pltpu.sync_copy(dst_hbm.at[wid], didx)

    @pl.when(sid < NS - 1)
    def _():
        r0 = pl.multiple_of(sid * RA, 8)
        pltpu.sync_copy(z_hbm, acc.at[pl.ds(r0, RA)])

    @pl.when(sid == NS - 1)
    def _():
        pltpu.sync_copy(z_hbm.at[pl.ds(0, RB)], acc.at[pl.ds(N - RB, RB)])

    plsc.subcore_barrier()

    def body(i, carry):
        pltpu.async_copy(g_hbm.at[sidx.at[i]], rows, sem).wait()
        pltpu.sync_copy(rows, acc.at[didx.at[i]], add=True)
        return carry

    lax.fori_loop(0, ITERS, body, 0)
    plsc.subcore_barrier()

    @pl.when(sid < NS - 1)
    def _():
        r0 = pl.multiple_of(sid * RA, 8)
        o0 = pl.multiple_of(cid * N + sid * RA, 8)
        pltpu.sync_copy(acc.at[pl.ds(r0, RA)], out_hbm.at[pl.ds(o0, RA)])

    @pl.when(sid == NS - 1)
    def _():
        o0 = pl.multiple_of(cid * N + (N - RB), 8)
        pltpu.sync_copy(acc.at[pl.ds(N - RB, RB)], out_hbm.at[pl.ds(o0, RB)])


def kernel(feat, edge_index, W1, b1, W2, b2):
    src = edge_index[0].reshape(NW, ITERS, CHUNK)
    dst = edge_index[1].reshape(NW, ITERS, CHUNK)
    zeros_slab = jnp.zeros((RA, D), jnp.float32)
    zrow = jnp.zeros((1, D), jnp.float32)
    b1row = b1.reshape(1, D)
    b2row = b2.reshape(1, D)

    g = _mlp(feat, W1, W2)
    p = _spmm(src, dst, g, zeros_slab)
    h1 = _combine(p, zrow, W2, zrow)
    p2 = _spmm(src, dst, h1, zeros_slab)
    return _combine(p2, b1row, W2, b2row)


# speedup vs baseline: 11.3436x; 1.6918x over previous
import functools

import jax
import jax.numpy as jnp
from jax import lax
from jax.experimental import pallas as pl
from jax.experimental.pallas import tpu as pltpu
from jax.experimental.pallas import tpu_sc as plsc

N = 10000
E = 320000
D = 128

NC = 2
NS = 16
NW = NC * NS
CHUNK = 125
EPT = E // NW
ITERS = EPT // CHUNK
STAGES = 2
SITERS = ITERS // STAGES
PAIRS = SITERS // 2
RA = 632
RB = N - (NS - 1) * RA


def _mlp_body(x_ref, w1_ref, w2_ref, o_ref):
    h = lax.dot_general(x_ref[...], w1_ref[...], (((1,), (1,)), ((), ())),
                        preferred_element_type=jnp.float32)
    o_ref[...] = lax.dot_general(h, w2_ref[...], (((1,), (1,)), ((), ())),
                                 preferred_element_type=jnp.float32)


def _mlp(x, W1, W2):
    return pl.pallas_call(
        _mlp_body,
        out_shape=jax.ShapeDtypeStruct((N, D), jnp.float32),
    )(x, W1, W2)


def _combine_body(p_ref, b1_ref, w2_ref, b2_ref, o_ref):
    c = lax.dot_general(b1_ref[...], w2_ref[...], (((1,), (1,)), ((), ())),
                        preferred_element_type=jnp.float32) + b2_ref[...]
    o_ref[...] = p_ref[:N] + p_ref[N:] + c


def _combine(p, b1row, W2, b2row):
    return pl.pallas_call(
        _combine_body,
        out_shape=jax.ShapeDtypeStruct((N, D), jnp.float32),
    )(p, b1row, W2, b2row)


_MESH = plsc.VectorSubcoreMesh(core_axis_name="c", subcore_axis_name="s")


@functools.partial(
    pl.kernel,
    mesh=_MESH,
    out_type=jax.ShapeDtypeStruct((NC * N, D), jnp.float32),
    scratch_types=[
        pltpu.VMEM((SITERS, CHUNK), jnp.int32),
        pltpu.VMEM((SITERS, CHUNK), jnp.int32),
        pltpu.VMEM((CHUNK, D), jnp.float32),
        pltpu.VMEM((CHUNK, D), jnp.float32),
        pltpu.VMEM_SHARED((N, D), jnp.float32),
        pltpu.SemaphoreType.DMA,
        pltpu.SemaphoreType.DMA,
    ],
)
def _spmm(src_hbm, dst_hbm, g_hbm, z_hbm, out_hbm,
          sidx, didx, rows0, rows1, acc, sem0, sem1):
    cid = lax.axis_index("c")
    sid = lax.axis_index("s")
    wid = cid * NS + sid

    @pl.when(sid < NS - 1)
    def _():
        r0 = pl.multiple_of(sid * RA, 8)
        pltpu.sync_copy(z_hbm, acc.at[pl.ds(r0, RA)])

    @pl.when(sid == NS - 1)
    def _():
        pltpu.sync_copy(z_hbm.at[pl.ds(0, RB)], acc.at[pl.ds(N - RB, RB)])

    plsc.subcore_barrier()

    for s in range(STAGES):
        pltpu.sync_copy(src_hbm.at[wid, pl.ds(s * SITERS, SITERS)], sidx)
        pltpu.sync_copy(dst_hbm.at[wid, pl.ds(s * SITERS, SITERS)], didx)
        pltpu.async_copy(g_hbm.at[sidx.at[0]], rows0, sem0)

        def body(p, carry):
            i0 = 2 * p
            pltpu.async_copy(g_hbm.at[sidx.at[i0 + 1]], rows1, sem1)
            pltpu.make_async_copy(g_hbm.at[sidx.at[i0]], rows0, sem0).wait()
            pltpu.sync_copy(rows0, acc.at[didx.at[i0]], add=True)

            @pl.when(p < PAIRS - 1)
            def _():
                pltpu.async_copy(g_hbm.at[sidx.at[i0 + 2]], rows0, sem0)

            pltpu.make_async_copy(g_hbm.at[sidx.at[i0 + 1]], rows1, sem1).wait()
            pltpu.sync_copy(rows1, acc.at[didx.at[i0 + 1]], add=True)
            return carry

        lax.fori_loop(0, PAIRS, body, 0)
    plsc.subcore_barrier()

    @pl.when(sid < NS - 1)
    def _():
        r0 = pl.multiple_of(sid * RA, 8)
        o0 = pl.multiple_of(cid * N + sid * RA, 8)
        pltpu.sync_copy(acc.at[pl.ds(r0, RA)], out_hbm.at[pl.ds(o0, RA)])

    @pl.when(sid == NS - 1)
    def _():
        o0 = pl.multiple_of(cid * N + (N - RB), 8)
        pltpu.sync_copy(acc.at[pl.ds(N - RB, RB)], out_hbm.at[pl.ds(o0, RB)])


def kernel(feat, edge_index, W1, b1, W2, b2):
    src = edge_index[0].reshape(NW, ITERS, CHUNK)
    dst = edge_index[1].reshape(NW, ITERS, CHUNK)
    zeros_slab = jnp.zeros((RA, D), jnp.float32)
    zrow = jnp.zeros((1, D), jnp.float32)
    b1row = b1.reshape(1, D)
    b2row = b2.reshape(1, D)

    g = _mlp(feat, W1, W2)
    p = _spmm(src, dst, g, zeros_slab)
    h1 = _combine(p, zrow, W2, zrow)
    p2 = _spmm(src, dst, h1, zeros_slab)
    return _combine(p2, b1row, W2, b2row)


# speedup vs baseline: 11.6104x; 1.0235x over previous
import functools

import jax
import jax.numpy as jnp
from jax import lax
from jax.experimental import pallas as pl
from jax.experimental.pallas import tpu as pltpu
from jax.experimental.pallas import tpu_sc as plsc

N = 10000
E = 320000
D = 128

NC = 2
NS = 16
NW = NC * NS
CHUNK = 125
EPT = E // NW
ITERS = EPT // CHUNK
STAGES = 2
SITERS = ITERS // STAGES
PAIRS = SITERS // 2
RA = 632
RB = N - (NS - 1) * RA


def _combine_body(p_ref, o_ref):
    o_ref[...] = p_ref[:N] + p_ref[N:]


def _combine(p):
    return pl.pallas_call(
        _combine_body,
        out_shape=jax.ShapeDtypeStruct((N, D), jnp.float32),
    )(p)


def _combine_mlp_body(p_ref, w1_ref, b1_ref, w2_ref, b2_ref, o_ref):
    h = p_ref[:N] + p_ref[N:]
    h = lax.dot_general(h, w1_ref[...], (((1,), (1,)), ((), ())),
                        preferred_element_type=jnp.float32) + b1_ref[...]
    o_ref[...] = lax.dot_general(h, w2_ref[...], (((1,), (1,)), ((), ())),
                                 preferred_element_type=jnp.float32) + b2_ref[...]


def _combine_mlp(p, W1, b1row, W2, b2row):
    return pl.pallas_call(
        _combine_mlp_body,
        out_shape=jax.ShapeDtypeStruct((N, D), jnp.float32),
    )(p, W1, b1row, W2, b2row)


_MESH = plsc.VectorSubcoreMesh(core_axis_name="c", subcore_axis_name="s")


@functools.partial(
    pl.kernel,
    mesh=_MESH,
    out_type=jax.ShapeDtypeStruct((NC * N, D), jnp.float32),
    scratch_types=[
        pltpu.VMEM((SITERS, CHUNK), jnp.int32),
        pltpu.VMEM((SITERS, CHUNK), jnp.int32),
        pltpu.VMEM((CHUNK, D), jnp.float32),
        pltpu.VMEM((CHUNK, D), jnp.float32),
        pltpu.VMEM_SHARED((N, D), jnp.float32),
        pltpu.SemaphoreType.DMA,
        pltpu.SemaphoreType.DMA,
    ],
)
def _spmm(src_hbm, dst_hbm, g_hbm, z_hbm, out_hbm,
          sidx, didx, rows0, rows1, acc, sem0, sem1):
    cid = lax.axis_index("c")
    sid = lax.axis_index("s")
    wid = cid * NS + sid

    @pl.when(sid < NS - 1)
    def _():
        r0 = pl.multiple_of(sid * RA, 8)
        pltpu.sync_copy(z_hbm, acc.at[pl.ds(r0, RA)])

    @pl.when(sid == NS - 1)
    def _():
        pltpu.sync_copy(z_hbm.at[pl.ds(0, RB)], acc.at[pl.ds(N - RB, RB)])

    plsc.subcore_barrier()

    for s in range(STAGES):
        pltpu.sync_copy(src_hbm.at[wid, pl.ds(s * SITERS, SITERS)], sidx)
        pltpu.sync_copy(dst_hbm.at[wid, pl.ds(s * SITERS, SITERS)], didx)
        pltpu.async_copy(g_hbm.at[sidx.at[0]], rows0, sem0)

        def body(p, carry):
            i0 = 2 * p
            pltpu.async_copy(g_hbm.at[sidx.at[i0 + 1]], rows1, sem1)
            pltpu.make_async_copy(g_hbm.at[sidx.at[i0]], rows0, sem0).wait()
            pltpu.sync_copy(rows0, acc.at[didx.at[i0]], add=True)

            @pl.when(p < PAIRS - 1)
            def _():
                pltpu.async_copy(g_hbm.at[sidx.at[i0 + 2]], rows0, sem0)

            pltpu.make_async_copy(g_hbm.at[sidx.at[i0 + 1]], rows1, sem1).wait()
            pltpu.sync_copy(rows1, acc.at[didx.at[i0 + 1]], add=True)
            return carry

        lax.fori_loop(0, PAIRS, body, 0)
    plsc.subcore_barrier()

    @pl.when(sid < NS - 1)
    def _():
        r0 = pl.multiple_of(sid * RA, 8)
        o0 = pl.multiple_of(cid * N + sid * RA, 8)
        pltpu.sync_copy(acc.at[pl.ds(r0, RA)], out_hbm.at[pl.ds(o0, RA)])

    @pl.when(sid == NS - 1)
    def _():
        o0 = pl.multiple_of(cid * N + (N - RB), 8)
        pltpu.sync_copy(acc.at[pl.ds(N - RB, RB)], out_hbm.at[pl.ds(o0, RB)])


def kernel(feat, edge_index, W1, b1, W2, b2):
    src = edge_index[0].reshape(NW, ITERS, CHUNK)
    dst = edge_index[1].reshape(NW, ITERS, CHUNK)
    zeros_slab = jnp.zeros((RA, D), jnp.float32)
    b1row = b1.reshape(1, D)
    b2row = b2.reshape(1, D)

    p = _spmm(src, dst, feat, zeros_slab)
    h1 = _combine(p)
    p2 = _spmm(src, dst, h1, zeros_slab)
    return _combine_mlp(p2, W1, b1row, W2, b2row)
